# baseline (speedup 1.0000x reference)
import jax
import jax.numpy as jnp
from jax.experimental import pallas as pl
from jax.experimental.pallas import tpu as pltpu

_NUM_CHANNELS = 5
_MEAN_SCALE = float(_NUM_CHANNELS) * (1.0 / _NUM_CHANNELS)

_EG = 25


def _fused_kernel(x_ref, e_ref, u_ref, xo_ref, uo_ref, eo_ref, scratch, sems):
    g = pl.program_id(0)
    j = pl.program_id(1)
    eg = pl.num_programs(1)
    ng = pl.num_programs(0)
    s = g * eg + j
    p = jax.lax.rem(s, 2)
    w = e_ref.shape[1]
    ecols = eg * w

    xo_ref[...] = x_ref[...] * _MEAN_SCALE

    @pl.when(s == 0)
    def _():
        uo_ref[...] = u_ref[...] * _MEAN_SCALE

    def row_copy(slot, gg, jj, k):
        base = (8 * gg + k) * ecols + jj * w
        return pltpu.make_async_copy(
            scratch.at[slot, k],
            eo_ref.at[0, pl.ds(base, w)],
            sems.at[slot, k],
        )

    @pl.when(s >= 2)
    def _():
        g2 = (s - 2) // eg
        j2 = jax.lax.rem(s - 2, eg)
        for k in range(8):
            row_copy(p, g2, j2, k).wait()

    scratch[p] = e_ref[...] * _MEAN_SCALE
    for k in range(8):
        row_copy(p, g, j, k).start()

    @pl.when(s == ng * eg - 1)
    def _():
        g1 = (s - 1) // eg
        j1 = jax.lax.rem(s - 1, eg)
        for k in range(8):
            row_copy(1 - p, g1, j1, k).wait()
        for k in range(8):
            row_copy(p, g, j, k).wait()


def kernel(x, edge_index, edge_attr, u, batch):
    del edge_index, batch
    n, d = x.shape
    e, de = edge_attr.shape

    et = jnp.transpose(edge_attr)
    ng = de // 8
    w = e // _EG
    nsteps = ng * _EG
    xb = n // nsteps

    x_m, u_m, e_m = pl.pallas_call(
        _fused_kernel,
        grid=(ng, _EG),
        in_specs=[
            pl.BlockSpec((xb, d), lambda g, j, eg=_EG: (g * eg + j, 0)),
            pl.BlockSpec((8, w), lambda g, j: (g, j)),
            pl.BlockSpec((1, d), lambda g, j: (0, 0)),
        ],
        out_specs=[
            pl.BlockSpec((xb, d), lambda g, j, eg=_EG: (g * eg + j, 0)),
            pl.BlockSpec((1, d), lambda g, j: (0, 0)),
            pl.BlockSpec(memory_space=pl.ANY),
        ],
        out_shape=[
            jax.ShapeDtypeStruct((n, d), x.dtype),
            jax.ShapeDtypeStruct((1, d), u.dtype),
            jax.ShapeDtypeStruct((1, de * e), edge_attr.dtype),
        ],
        scratch_shapes=[
            pltpu.VMEM((2, 8, w), edge_attr.dtype),
            pltpu.SemaphoreType.DMA((2, 8)),
        ],
    )(x, et, u)

    return (
        x_m[:, :, None],
        e_m.reshape(de, e, 1).transpose(1, 0, 2),
        u_m[:, :, None],
    )


# speedup vs baseline: 1.9309x; 1.9309x over previous
import jax
import jax.numpy as jnp
from jax.experimental import pallas as pl
from jax.experimental.pallas import tpu as pltpu

_NUM_CHANNELS = 5
_MEAN_SCALE = float(_NUM_CHANNELS) * (1.0 / _NUM_CHANNELS)

_EG = 25


def _fused_kernel(x_ref, e_ref, u_ref, xo_ref, uo_ref, eo_ref, scratch, sems):
    g = pl.program_id(0)
    j = pl.program_id(1)
    eg = pl.num_programs(1)
    ng = pl.num_programs(0)
    s = g * eg + j
    p = jax.lax.rem(s, 3)
    w = e_ref.shape[1]
    ecols = eg * w

    xo_ref[...] = x_ref[...] * _MEAN_SCALE

    @pl.when(s == 0)
    def _():
        uo_ref[...] = u_ref[...] * _MEAN_SCALE

    def row_copy(slot, gg, jj, k):
        base = (8 * gg + k) * ecols + jj * w
        return pltpu.make_async_copy(
            scratch.at[slot, k],
            eo_ref.at[0, pl.ds(base, w)],
            sems.at[slot, k],
        )

    @pl.when(s >= 3)
    def _():
        g2 = (s - 3) // eg
        j2 = jax.lax.rem(s - 3, eg)
        for k in range(8):
            row_copy(p, g2, j2, k).wait()

    scratch[p] = e_ref[...] * _MEAN_SCALE
    for k in range(8):
        row_copy(p, g, j, k).start()

    @pl.when(s == ng * eg - 1)
    def _():
        for back in (2, 1, 0):
            sb = s - back
            gb = sb // eg
            jb = jax.lax.rem(sb, eg)
            for k in range(8):
                row_copy(jax.lax.rem(sb, 3), gb, jb, k).wait()


def kernel(x, edge_index, edge_attr, u, batch):
    del edge_index, batch
    n, d = x.shape
    e, de = edge_attr.shape

    et = jnp.transpose(edge_attr)
    ng = de // 8
    w = e // _EG
    nsteps = ng * _EG
    xb = n // nsteps
    x3 = x.reshape(nsteps, xb, d)

    x_m, u_m, e_m = pl.pallas_call(
        _fused_kernel,
        grid=(ng, _EG),
        in_specs=[
            pl.BlockSpec((1, xb, d), lambda g, j, eg=_EG: (g * eg + j, 0, 0)),
            pl.BlockSpec((8, w), lambda g, j: (g, j)),
            pl.BlockSpec((1, d), lambda g, j: (0, 0)),
        ],
        out_specs=[
            pl.BlockSpec((1, xb, d), lambda g, j, eg=_EG: (g * eg + j, 0, 0)),
            pl.BlockSpec((1, d), lambda g, j: (0, 0)),
            pl.BlockSpec(memory_space=pl.ANY),
        ],
        out_shape=[
            jax.ShapeDtypeStruct((nsteps, xb, d), x.dtype),
            jax.ShapeDtypeStruct((1, d), u.dtype),
            jax.ShapeDtypeStruct((1, de * e), edge_attr.dtype),
        ],
        scratch_shapes=[
            pltpu.VMEM((3, 8, w), edge_attr.dtype),
            pltpu.SemaphoreType.DMA((3, 8)),
        ],
    )(x3, et, u)

    return (
        x_m.reshape(n, d)[:, :, None],
        e_m.reshape(de, e, 1).transpose(1, 0, 2),
        u_m[:, :, None],
    )


# speedup vs baseline: 1.9436x; 1.0066x over previous
import jax
import jax.numpy as jnp
from jax.experimental import pallas as pl
from jax.experimental.pallas import tpu as pltpu

_NUM_CHANNELS = 5
_MEAN_SCALE = float(_NUM_CHANNELS) * (1.0 / _NUM_CHANNELS)

_EG = 5


def _fused_kernel(x_ref, e_ref, u_ref, xo_ref, uo_ref, eo_ref, scratch, sems):
    g = pl.program_id(0)
    j = pl.program_id(1)
    eg = pl.num_programs(1)
    ng = pl.num_programs(0)
    s = g * eg + j
    p = jax.lax.rem(s, 2)
    w = e_ref.shape[1]
    ecols = eg * w

    xo_ref[...] = x_ref[...] * _MEAN_SCALE

    @pl.when(s == 0)
    def _():
        uo_ref[...] = u_ref[...] * _MEAN_SCALE

    def row_copy(slot, gg, jj, k):
        base = (8 * gg + k) * ecols + jj * w
        return pltpu.make_async_copy(
            scratch.at[slot, k],
            eo_ref.at[0, pl.ds(base, w)],
            sems.at[slot, k],
        )

    @pl.when(s >= 2)
    def _():
        g2 = (s - 2) // eg
        j2 = jax.lax.rem(s - 2, eg)
        for k in range(8):
            row_copy(p, g2, j2, k).wait()

    scratch[p] = e_ref[...] * _MEAN_SCALE
    for k in range(8):
        row_copy(p, g, j, k).start()

    @pl.when(s == ng * eg - 1)
    def _():
        g1 = (s - 1) // eg
        j1 = jax.lax.rem(s - 1, eg)
        for k in range(8):
            row_copy(1 - p, g1, j1, k).wait()
        for k in range(8):
            row_copy(p, g, j, k).wait()


def kernel(x, edge_index, edge_attr, u, batch):
    del edge_index, batch
    n, d = x.shape
    e, de = edge_attr.shape

    et = jnp.transpose(edge_attr)
    ng = de // 8
    w = e // _EG
    nsteps = ng * _EG
    xb = n // nsteps

    x_m, u_m, e_m = pl.pallas_call(
        _fused_kernel,
        grid=(ng, _EG),
        in_specs=[
            pl.BlockSpec((xb, d), lambda g, j, eg=_EG: (g * eg + j, 0)),
            pl.BlockSpec((8, w), lambda g, j: (g, j)),
            pl.BlockSpec((1, d), lambda g, j: (0, 0)),
        ],
        out_specs=[
            pl.BlockSpec((xb, d), lambda g, j, eg=_EG: (g * eg + j, 0)),
            pl.BlockSpec((1, d), lambda g, j: (0, 0)),
            pl.BlockSpec(memory_space=pl.ANY),
        ],
        out_shape=[
            jax.ShapeDtypeStruct((n, d), x.dtype),
            jax.ShapeDtypeStruct((1, d), u.dtype),
            jax.ShapeDtypeStruct((1, de * e), edge_attr.dtype),
        ],
        scratch_shapes=[
            pltpu.VMEM((2, 8, w), edge_attr.dtype),
            pltpu.SemaphoreType.DMA((2, 8)),
        ],
    )(x, et, u)

    return (
        x_m[:, :, None],
        e_m.reshape(de, e, 1).transpose(1, 0, 2),
        u_m[:, :, None],
    )
